# baseline (speedup 1.0000x reference)
import functools

import jax
import jax.numpy as jnp
from jax import lax
from jax.experimental import pallas as pl
from jax.experimental.pallas import tpu as pltpu
from jax.experimental.pallas import tpu_sc as plsc

F = 6
N = F * F * F
KEEP = 108
LANES = 16
NV_TC = 14
NPAD_TC = NV_TC * LANES
NV_SC = 15
NPAD_SC = NV_SC * LANES
PAD_KEY = 0x7FFFFFFF

KS0 = 0
KS1 = 42
KS2 = KS0 ^ KS1 ^ 0x1BD11BDA
ROUNDS = ((13, 15, 26, 6), (17, 29, 16, 24))


def _rotl(x, d):
    return lax.shift_left(x, d) | lax.shift_right_logical(x, 32 - d)


def _threefry(x0, x1):
    ks = (KS0, KS1, KS2)
    x0 = x0 + KS0
    x1 = x1 + KS1
    for i in range(5):
        for r in ROUNDS[i % 2]:
            x0 = x0 + x1
            x1 = _rotl(x1, r)
            x1 = x1 ^ x0
        x0 = x0 + ks[(i + 1) % 3]
        x1 = x1 + (ks[(i + 2) % 3] + i + 1)
    return x0, x1


@functools.lru_cache(maxsize=None)
def _build_sc(rows_sc):
    info = plsc.get_sparse_core_info()
    nw = info.num_cores * info.num_subcores
    rows_per_w = rows_sc // nw

    mesh = plsc.VectorSubcoreMesh(core_axis_name="c", subcore_axis_name="s")

    @functools.partial(
        pl.kernel,
        out_type=jax.ShapeDtypeStruct((rows_sc, NPAD_SC), jnp.int32),
        mesh=mesh,
        scratch_types=[
            pltpu.VMEM((rows_per_w, NPAD_SC), jnp.int32),
            pltpu.VMEM((NPAD_SC,), jnp.int32),
            pltpu.VMEM((LANES,), jnp.int32),
            pltpu.VMEM((LANES,), jnp.int32),
        ],
        compiler_params=pltpu.CompilerParams(needs_layout_passes=False),
    )
    def mask_kernel(out_hbm, mbuf, kbuf, hist, cums):
        wid = lax.axis_index("s") * info.num_cores + lax.axis_index("c")
        base = wid * rows_per_w
        lane = lax.iota(jnp.int32, LANES)
        zero_v = jnp.zeros((LANES,), jnp.int32)
        ones_v = jnp.full((LANES,), 1, jnp.int32)

        def row_body(t, carry):
            row = base + t

            def tf3(i, c2):
                for j in range(3):
                    col = lane + (i * 3 + j) * LANES
                    flat = row * N + col
                    b0, b1 = _threefry(zero_v, flat)
                    bits = b0 ^ b1
                    key = (
                        lax.shift_left(lax.shift_right_logical(bits, 9), 8)
                        | col
                    )
                    key = jnp.where(col < N, key, PAD_KEY)
                    kbuf[pl.ds((i * 3 + j) * LANES, LANES)] = key
                return c2

            lax.fori_loop(0, 5, tf3, jnp.int32(0))

            def phase(p, c2):
                prefix, target = c2
                sh = jnp.broadcast_to(28 - 4 * p, (LANES,))
                hist[...] = zero_v
                pref_hi = lax.shift_right_logical(
                    lax.shift_right_logical(prefix, sh), 4
                )
                for v in range(NV_SC):
                    k = kbuf[pl.ds(v * LANES, LANES)]
                    ksh = lax.shift_right_logical(k, sh)
                    nib = ksh & 15
                    act = lax.shift_right_logical(ksh, 4) == pref_hi
                    plsc.addupdate_scatter(hist, [nib], ones_v, mask=act)
                c = plsc.cumsum(hist[...])
                bstar = plsc.all_reduce_population_count(c <= target)
                cums[...] = c
                cb = plsc.load_gather(cums, [jnp.maximum(bstar - 1, 0)])
                target = target - jnp.where(bstar > 0, cb, 0)
                prefix = prefix | lax.shift_left(bstar, sh)
                return (prefix, target)

            thr_v, _ = lax.fori_loop(
                0, 8, phase,
                (zero_v, jnp.full((LANES,), KEEP - 1, jnp.int32)),
            )

            for v in range(NV_SC):
                k = kbuf[pl.ds(v * LANES, LANES)]
                mbuf[t, pl.ds(v * LANES, LANES)] = jnp.where(
                    k <= thr_v, 1, 0
                )
            return carry

        lax.fori_loop(0, rows_per_w, row_body, jnp.int32(0))
        pltpu.sync_copy(mbuf, out_hbm.at[pl.ds(base, rows_per_w), :])

    return mask_kernel


@functools.lru_cache(maxsize=None)
def _build_tc(base_row, rows):
    def body(out_ref):
        c = lax.broadcasted_iota(jnp.int32, (NPAD_TC, rows), 0)
        r = lax.broadcasted_iota(jnp.int32, (NPAD_TC, rows), 1) + base_row
        flat = r * N + c
        x0, x1 = _threefry(jnp.zeros((NPAD_TC, rows), jnp.int32), flat)
        bits = x0 ^ x1
        key = lax.shift_left(lax.shift_right_logical(bits, 9), 8) | c
        key = jnp.where(c < N, key, PAD_KEY)

        top_bit = jnp.full((1, rows), 1 << 30, jnp.int32)

        def bit_step(i, x):
            cand = x | lax.shift_right_logical(top_bit, i)
            cnt = jnp.sum((key < cand).astype(jnp.int32), axis=0, keepdims=True)
            return jnp.where(cnt < KEEP, cand, x)

        thr = lax.fori_loop(0, 31, bit_step, jnp.zeros((1, rows), jnp.int32))
        out_ref[...] = jnp.where(key <= thr, 1, 0)

    return pl.pallas_call(
        body, out_shape=jax.ShapeDtypeStruct((NPAD_TC, rows), jnp.int32)
    )


ROWS_SC = 256


def kernel(input):
    batch = input.shape[0]
    sc_raw = _build_sc(ROWS_SC)()
    tc_raw = _build_tc(ROWS_SC, batch - ROWS_SC)().T
    raw = jnp.concatenate([sc_raw[:, :N], tc_raw[:, :N]], axis=0)
    return (raw != 0).reshape(batch, 1, F, F, F)


# speedup vs baseline: 1.2805x; 1.2805x over previous
import functools

import jax
import jax.numpy as jnp
from jax import lax
from jax.experimental import pallas as pl
from jax.experimental.pallas import tpu as pltpu
from jax.experimental.pallas import tpu_sc as plsc

F = 6
N = F * F * F
KEEP = 108
LANES = 16
NV_TC = 14
NPAD_TC = NV_TC * LANES
NV_SC = 14
NPAD_SC = NV_SC * LANES
PAD_KEY = 0x7FFFFFFF

KS0 = 0
KS1 = 42
KS2 = KS0 ^ KS1 ^ 0x1BD11BDA
ROUNDS = ((13, 15, 26, 6), (17, 29, 16, 24))


def _rotl(x, d):
    return lax.shift_left(x, d) | lax.shift_right_logical(x, 32 - d)


def _threefry(x0, x1):
    ks = (KS0, KS1, KS2)
    x0 = x0 + KS0
    x1 = x1 + KS1
    for i in range(5):
        for r in ROUNDS[i % 2]:
            x0 = x0 + x1
            x1 = _rotl(x1, r)
            x1 = x1 ^ x0
        x0 = x0 + ks[(i + 1) % 3]
        x1 = x1 + (ks[(i + 2) % 3] + i + 1)
    return x0, x1


@functools.lru_cache(maxsize=None)
def _build_sc(rows_sc):
    info = plsc.get_sparse_core_info()
    nw = info.num_cores * info.num_subcores
    rows_per_w = rows_sc // nw

    mesh = plsc.VectorSubcoreMesh(core_axis_name="c", subcore_axis_name="s")

    @functools.partial(
        pl.kernel,
        out_type=jax.ShapeDtypeStruct((rows_sc, NPAD_SC), jnp.int32),
        mesh=mesh,
        scratch_types=[
            pltpu.VMEM((rows_per_w, NPAD_SC), jnp.int32),
            pltpu.VMEM((LANES,), jnp.int32),
            pltpu.VMEM((LANES,), jnp.int32),
        ],
        compiler_params=pltpu.CompilerParams(needs_layout_passes=False),
    )
    def mask_kernel(out_hbm, mbuf, hist, cums):
        wid = lax.axis_index("s") * info.num_cores + lax.axis_index("c")
        base = wid * rows_per_w
        lane = lax.iota(jnp.int32, LANES)
        zero_v = jnp.zeros((LANES,), jnp.int32)
        ones_v = jnp.full((LANES,), 1, jnp.int32)

        def row_body(t, carry):
            row = base + t
            keys = []
            for v in range(NV_SC):
                col = lane + (v * LANES)
                flat = row * N + col
                b0, b1 = _threefry(zero_v, flat)
                bits = b0 ^ b1
                key = lax.shift_left(lax.shift_right_logical(bits, 9), 8) | col
                if v == NV_SC - 1:
                    key = jnp.where(col < N, key, PAD_KEY)
                keys.append(key)

            prefix = zero_v
            target = jnp.full((LANES,), KEEP - 1, jnp.int32)
            for sh in range(28, -1, -4):
                hist[...] = zero_v
                if sh == 28:
                    for k in keys:
                        nib = lax.shift_right_logical(k, sh)
                        plsc.addupdate_scatter(hist, [nib], ones_v)
                else:
                    pref_hi = lax.shift_right_logical(prefix, sh + 4)
                    for k in keys:
                        nib = lax.shift_right_logical(k, sh) & 15
                        act = lax.shift_right_logical(k, sh + 4) == pref_hi
                        plsc.addupdate_scatter(hist, [nib], ones_v, mask=act)
                c = plsc.cumsum(hist[...])
                bstar = plsc.all_reduce_population_count(c <= target)
                cums[...] = c
                cb = plsc.load_gather(cums, [jnp.maximum(bstar - 1, 0)])
                target = target - jnp.where(bstar > 0, cb, 0)
                prefix = prefix | lax.shift_left(bstar, sh)
            thr_v = prefix

            for v in range(NV_SC):
                mbuf[t, pl.ds(v * LANES, LANES)] = jnp.where(
                    keys[v] <= thr_v, 1, 0
                )
            return carry

        lax.fori_loop(0, rows_per_w, row_body, jnp.int32(0))
        pltpu.sync_copy(mbuf, out_hbm.at[pl.ds(base, rows_per_w), :])

    return mask_kernel


@functools.lru_cache(maxsize=None)
def _build_tc(base_row, rows):
    def body(out_ref):
        c = lax.broadcasted_iota(jnp.int32, (N, rows), 0)
        r = lax.broadcasted_iota(jnp.int32, (N, rows), 1) + base_row
        flat = r * N + c
        x0, x1 = _threefry(jnp.zeros((N, rows), jnp.int32), flat)
        bits = x0 ^ x1
        key = lax.shift_left(lax.shift_right_logical(bits, 9), 8) | c

        top_bit = jnp.full((1, rows), 1 << 30, jnp.int32)

        def bit_step(i, x):
            cand = x | lax.shift_right_logical(top_bit, i)
            cnt = jnp.sum((key < cand).astype(jnp.int32), axis=0, keepdims=True)
            return jnp.where(cnt < KEEP, cand, x)

        thr = lax.fori_loop(0, 31, bit_step, jnp.zeros((1, rows), jnp.int32))
        out_ref[...] = key <= thr

    return pl.pallas_call(
        body, out_shape=jax.ShapeDtypeStruct((N, rows), jnp.bool_)
    )


ROWS_SC = 256


def kernel(input):
    batch = input.shape[0]
    sc_raw = _build_sc(ROWS_SC)()
    tc_mask = _build_tc(ROWS_SC, batch - ROWS_SC)().T
    sc_mask = sc_raw[:, :N] != 0
    return jnp.concatenate([sc_mask, tc_mask], axis=0).reshape(
        batch, 1, F, F, F
    )
